# Initial kernel scaffold; baseline (speedup 1.0000x reference)
import jax
import jax.numpy as jnp
from jax.experimental import pallas as pl


def kernel(adj_indices, adj_values, user_embedding, item_embedding, filter_w):
    raise NotImplementedError("write your pallas kernel here")


# speedup vs baseline: 4.0176x; 4.0176x over previous
import functools

import jax
import jax.numpy as jnp
from jax import lax
from jax.experimental import pallas as pl
from jax.experimental.pallas import tpu as pltpu
from jax.experimental.pallas import tpu_sc as plsc

N_USERS = 5000
N_ITEMS = 5000
EMB = 128
N = N_USERS + N_ITEMS
E = 320000

NC = 2
NS = 16
NW = NC * NS
B = 128
NB = 80
CH = 8
NCH = NB // CH
N_PAD = 10240

ROWS_PER_TILE = N_PAD // NS
ROW_CHUNK = 128
N_ROW_CHUNKS = ROWS_PER_TILE // ROW_CHUNK


def _sc_spmm_kernel(emb0_hbm, row_hbm, col_hbm, val_hbm, out_hbm,
                    acc, rowr, colr, valr, gbuf0, gbuf1,
                    gsem0, gsem1, ssem0, ssem1):
    c = lax.axis_index("c")
    s = lax.axis_index("s")
    w = c * NS + s

    def zrow(i, _):
        for r in range(8):
            gbuf0[i, pl.ds(r * 16, 16)] = jnp.zeros((16,), jnp.float32)
        return 0
    lax.fori_loop(0, B, zrow, 0)
    base = s * ROWS_PER_TILE
    for j in range(N_ROW_CHUNKS):
        pltpu.sync_copy(gbuf0, acc.at[pl.ds(base + j * ROW_CHUNK, ROW_CHUNK)])

    pltpu.sync_copy(row_hbm.at[w, pl.ds(0, CH)], rowr.at[0])
    pltpu.sync_copy(col_hbm.at[w, pl.ds(0, CH)], colr.at[0])
    pltpu.sync_copy(val_hbm.at[w, pl.ds(0, CH)], valr.at[0])
    plsc.subcore_barrier()

    def scale(buf, par, k):
        def body(g, _):
            vrow = valr[par, k, pl.ds(g * 16, 16)]
            for j in range(16):
                kk = g * 16 + j
                v = vrow[j]
                for r in range(8):
                    sl = pl.ds(r * 16, 16)
                    buf[kk, sl] = buf[kk, sl] * v
            return 0
        lax.fori_loop(0, B // 16, body, 0)

    def slot(b):
        return lax.rem(lax.div(b, CH), 2), lax.rem(b, CH)

    pltpu.async_copy(emb0_hbm.at[colr.at[0, 0]], gbuf0, gsem0)

    def batch_pair(g, _):
        b0 = 2 * g
        b1 = b0 + 1
        par, k0 = slot(b0)
        _, k1 = slot(b1)

        @pl.when(g > 0)
        def _():
            parp, kp = slot(b0 - 1)
            pltpu.make_async_copy(gbuf1, acc.at[rowr.at[parp, kp]], ssem1).wait()
        pltpu.async_copy(emb0_hbm.at[colr.at[par, k1]], gbuf1, gsem1)

        pltpu.make_async_copy(emb0_hbm.at[colr.at[par, k0]], gbuf0, gsem0).wait()
        scale(gbuf0, par, k0)
        pltpu.async_copy(gbuf0, acc.at[rowr.at[par, k0]], ssem0, add=True)

        @pl.when((lax.rem(g, CH // 2) == 0) & (b0 + CH < NB))
        def _():
            off = pl.multiple_of((lax.div(b0, CH) + 1) * CH, CH)
            pltpu.sync_copy(row_hbm.at[w, pl.ds(off, CH)], rowr.at[1 - par])
            pltpu.sync_copy(col_hbm.at[w, pl.ds(off, CH)], colr.at[1 - par])
            pltpu.sync_copy(val_hbm.at[w, pl.ds(off, CH)], valr.at[1 - par])

        pltpu.make_async_copy(emb0_hbm.at[colr.at[par, k1]], gbuf1, gsem1).wait()
        scale(gbuf1, par, k1)
        pltpu.make_async_copy(gbuf0, acc.at[rowr.at[par, k0]], ssem0).wait()

        @pl.when(g < NB // 2 - 1)
        def _():
            parn, kn = slot(b0 + 2)
            pltpu.async_copy(emb0_hbm.at[colr.at[parn, kn]], gbuf0, gsem0)
        pltpu.async_copy(gbuf1, acc.at[rowr.at[par, k1]], ssem1, add=True)
        return 0
    lax.fori_loop(0, NB // 2, batch_pair, 0)
    parl, kl = slot(NB - 1)
    pltpu.make_async_copy(gbuf1, acc.at[rowr.at[parl, kl]], ssem1).wait()

    plsc.subcore_barrier()

    for j in range(N_ROW_CHUNKS):
        r0 = base + j * ROW_CHUNK
        pltpu.sync_copy(acc.at[pl.ds(r0, ROW_CHUNK)], gbuf0)
        pltpu.sync_copy(gbuf0, out_hbm.at[c, pl.ds(r0, ROW_CHUNK)])


def _sc_spmm(emb0, rows, cols, vals):
    mesh = plsc.VectorSubcoreMesh(core_axis_name="c", subcore_axis_name="s")
    kfn = functools.partial(
        pl.kernel,
        mesh=mesh,
        out_type=jax.ShapeDtypeStruct((NC, N_PAD, EMB), jnp.float32),
        scratch_types=[
            pltpu.VMEM_SHARED((N_PAD, EMB), jnp.float32),
            pltpu.VMEM((2, CH, B), jnp.int32),
            pltpu.VMEM((2, CH, B), jnp.int32),
            pltpu.VMEM((2, CH, B), jnp.float32),
            pltpu.VMEM((B, EMB), jnp.float32),
            pltpu.VMEM((B, EMB), jnp.float32),
            pltpu.SemaphoreType.DMA,
            pltpu.SemaphoreType.DMA,
            pltpu.SemaphoreType.DMA,
            pltpu.SemaphoreType.DMA,
        ],
    )(_sc_spmm_kernel)
    return kfn(emb0, rows, cols, vals)


def _tc_filter_kernel(emb0_ref, acc_ref, w_ref, out_ref):
    e = emb0_ref[...]
    spmm = acc_ref[0] + acc_ref[1]
    x = 2.0 * e - spmm
    y = jax.nn.sigmoid(jnp.dot(x, w_ref[...], preferred_element_type=jnp.float32))
    out_ref[:, :EMB] = e
    out_ref[:, EMB:] = y


def _tc_filter(emb0, acc, filter_w):
    blk = 1024
    grid = N_PAD // blk
    return pl.pallas_call(
        _tc_filter_kernel,
        grid=(grid,),
        in_specs=[
            pl.BlockSpec((blk, EMB), lambda i: (i, 0)),
            pl.BlockSpec((NC, blk, EMB), lambda i: (0, i, 0)),
            pl.BlockSpec((EMB, EMB), lambda i: (0, 0)),
        ],
        out_specs=pl.BlockSpec((blk, 2 * EMB), lambda i: (i, 0)),
        out_shape=jax.ShapeDtypeStruct((N_PAD, 2 * EMB), jnp.float32),
    )(emb0, acc, filter_w)


@jax.jit
def kernel(adj_indices, adj_values, user_embedding, item_embedding, filter_w):
    pad_rows = jnp.zeros((N_PAD - N, EMB), jnp.float32)
    emb0 = jnp.concatenate([user_embedding, item_embedding, pad_rows], axis=0)

    e_pad = NW * NB * B
    row = jnp.pad(adj_indices[0], (0, e_pad - E)).reshape(NW, NB, B)
    col = jnp.pad(adj_indices[1], (0, e_pad - E)).reshape(NW, NB, B)
    val = jnp.pad(adj_values, (0, e_pad - E)).reshape(NW, NB, B)

    acc = _sc_spmm(emb0, row, col, val)
    all_emb = _tc_filter(emb0, acc, filter_w)
    return (all_emb[:N_USERS], all_emb[N_USERS:N])


# speedup vs baseline: 4.4512x; 1.1079x over previous
import functools

import jax
import jax.numpy as jnp
from jax import lax
from jax.experimental import pallas as pl
from jax.experimental.pallas import tpu as pltpu
from jax.experimental.pallas import tpu_sc as plsc

N_USERS = 5000
N_ITEMS = 5000
EMB = 128
N = N_USERS + N_ITEMS
E = 320000

NC = 2
NS = 16
NW = NC * NS
B = 128
NB0 = 152
NB1 = 8
NB_MAX = NB0
CH = 8
N_PAD = 10240

ROWS_PER_TILE = N_PAD // NS
ROW_CHUNK = 128
N_ROW_CHUNKS = ROWS_PER_TILE // ROW_CHUNK


def _sc_spmm_kernel(emb0_hbm, row_hbm, col_hbm, val_hbm, out_hbm,
                    acc, rowr, colr, valr, gbuf0, gbuf1,
                    gsem0, gsem1, ssem0, ssem1):
    c = lax.axis_index("c")
    s = lax.axis_index("s")
    w = c * NS + s
    nb = jnp.where(c == 0, NB0, NB1)

    def zrow(i, _):
        for r in range(8):
            gbuf0[i, pl.ds(r * 16, 16)] = jnp.zeros((16,), jnp.float32)
        return 0
    lax.fori_loop(0, B, zrow, 0)
    base = s * ROWS_PER_TILE
    for j in range(N_ROW_CHUNKS):
        pltpu.sync_copy(gbuf0, acc.at[pl.ds(base + j * ROW_CHUNK, ROW_CHUNK)])

    pltpu.sync_copy(row_hbm.at[w, pl.ds(0, CH)], rowr.at[0])
    pltpu.sync_copy(col_hbm.at[w, pl.ds(0, CH)], colr.at[0])
    pltpu.sync_copy(val_hbm.at[w, pl.ds(0, CH)], valr.at[0])
    plsc.subcore_barrier()

    def scale(buf, par, k):
        def body(g, _):
            vrow = valr[par, k, pl.ds(g * 16, 16)]
            for j in range(16):
                kk = g * 16 + j
                v = vrow[j]
                for r in range(8):
                    sl = pl.ds(r * 16, 16)
                    buf[kk, sl] = buf[kk, sl] * v
            return 0
        lax.fori_loop(0, B // 16, body, 0)

    def slot(b):
        return lax.rem(lax.div(b, CH), 2), lax.rem(b, CH)

    pltpu.async_copy(emb0_hbm.at[colr.at[0, 0]], gbuf0, gsem0)

    def batch_pair(g, _):
        b0 = 2 * g
        b1 = b0 + 1
        par, k0 = slot(b0)
        _, k1 = slot(b1)

        @pl.when(g > 0)
        def _():
            parp, kp = slot(b0 - 1)
            pltpu.make_async_copy(gbuf1, acc.at[rowr.at[parp, kp]], ssem1).wait()
        pltpu.async_copy(emb0_hbm.at[colr.at[par, k1]], gbuf1, gsem1)

        pltpu.make_async_copy(emb0_hbm.at[colr.at[par, k0]], gbuf0, gsem0).wait()
        scale(gbuf0, par, k0)
        pltpu.async_copy(gbuf0, acc.at[rowr.at[par, k0]], ssem0, add=True)

        @pl.when((lax.rem(g, CH // 2) == 0) & (b0 + CH < nb))
        def _():
            off = pl.multiple_of((lax.div(b0, CH) + 1) * CH, CH)
            pltpu.sync_copy(row_hbm.at[w, pl.ds(off, CH)], rowr.at[1 - par])
            pltpu.sync_copy(col_hbm.at[w, pl.ds(off, CH)], colr.at[1 - par])
            pltpu.sync_copy(val_hbm.at[w, pl.ds(off, CH)], valr.at[1 - par])

        pltpu.make_async_copy(emb0_hbm.at[colr.at[par, k1]], gbuf1, gsem1).wait()
        scale(gbuf1, par, k1)
        pltpu.make_async_copy(gbuf0, acc.at[rowr.at[par, k0]], ssem0).wait()

        @pl.when(g < nb // 2 - 1)
        def _():
            parn, kn = slot(b0 + 2)
            pltpu.async_copy(emb0_hbm.at[colr.at[parn, kn]], gbuf0, gsem0)
        pltpu.async_copy(gbuf1, acc.at[rowr.at[par, k1]], ssem1, add=True)
        return 0
    lax.fori_loop(0, nb // 2, batch_pair, 0)
    parl, kl = slot(nb - 1)
    pltpu.make_async_copy(gbuf1, acc.at[rowr.at[parl, kl]], ssem1).wait()

    plsc.subcore_barrier()

    for j in range(N_ROW_CHUNKS):
        r0 = base + j * ROW_CHUNK
        pltpu.sync_copy(acc.at[pl.ds(r0, ROW_CHUNK)], gbuf0)
        pltpu.sync_copy(gbuf0, out_hbm.at[c, pl.ds(r0, ROW_CHUNK)])


def _sc_spmm(emb0, rows, cols, vals):
    out_ref = jax.new_ref(jnp.zeros((NC, N_PAD, EMB), jnp.float32))
    mesh = plsc.VectorSubcoreMesh(core_axis_name="c", subcore_axis_name="s")
    kfn = functools.partial(
        pl.kernel,
        mesh=mesh,
        out_type=(),
        scratch_types=[
            pltpu.VMEM_SHARED((N_PAD, EMB), jnp.float32),
            pltpu.VMEM((2, CH, B), jnp.int32),
            pltpu.VMEM((2, CH, B), jnp.int32),
            pltpu.VMEM((2, CH, B), jnp.float32),
            pltpu.VMEM((B, EMB), jnp.float32),
            pltpu.VMEM((B, EMB), jnp.float32),
            pltpu.SemaphoreType.DMA,
            pltpu.SemaphoreType.DMA,
            pltpu.SemaphoreType.DMA,
            pltpu.SemaphoreType.DMA,
        ],
    )(_sc_spmm_kernel)
    kfn(emb0, rows, cols, vals, out_ref)
    return out_ref[...]


def _tc_filter_kernel(emb0_ref, acc_ref, w_ref, out_ref):
    e = emb0_ref[...]
    spmm = acc_ref[0] + acc_ref[1]
    x = 2.0 * e - spmm
    y = jax.nn.sigmoid(jnp.dot(x, w_ref[...], preferred_element_type=jnp.float32))
    out_ref[:, :EMB] = e
    out_ref[:, EMB:] = y


def _tc_filter(emb0, acc, filter_w):
    blk = 1024
    grid = N_PAD // blk
    return pl.pallas_call(
        _tc_filter_kernel,
        grid=(grid,),
        in_specs=[
            pl.BlockSpec((blk, EMB), lambda i: (i, 0)),
            pl.BlockSpec((NC, blk, EMB), lambda i: (0, i, 0)),
            pl.BlockSpec((EMB, EMB), lambda i: (0, 0)),
        ],
        out_specs=pl.BlockSpec((blk, 2 * EMB), lambda i: (i, 0)),
        out_shape=jax.ShapeDtypeStruct((N_PAD, 2 * EMB), jnp.float32),
    )(emb0, acc, filter_w)


@jax.jit
def kernel(adj_indices, adj_values, user_embedding, item_embedding, filter_w):
    pad_rows = jnp.zeros((N_PAD - N, EMB), jnp.float32)
    emb0 = jnp.concatenate([user_embedding, item_embedding, pad_rows], axis=0)

    e_pad = NS * (NB0 + NB1) * B
    e_c0 = NS * NB0 * B

    def _shard(a):
        a = jnp.pad(a, (0, e_pad - E))
        p0 = a[:e_c0].reshape(NS, NB0, B)
        p1 = jnp.pad(a[e_c0:].reshape(NS, NB1, B),
                     ((0, 0), (0, NB_MAX - NB1), (0, 0)))
        return jnp.concatenate([p0, p1], axis=0)

    row = _shard(adj_indices[0])
    col = _shard(adj_indices[1])
    val = _shard(adj_values)

    acc = _sc_spmm(emb0, row, col, val)
    all_emb = _tc_filter(emb0, acc, filter_w)
    return (all_emb[:N_USERS], all_emb[N_USERS:N])
